# baseline (speedup 1.0000x reference)
import functools

import jax
import jax.numpy as jnp
from jax import lax
from jax.experimental import pallas as pl
from jax.experimental.pallas import tpu as pltpu
from jax.experimental.pallas import tpu_sc as plsc

N = 10000
E = 320000
D_IN = 128
HIDDEN = 128
K = 10
L = 16
NP = 10240
SL = NP // 16
EPT = E // 16


def _proj_body(x_ref, wi_ref, bi_ref, w8_ref, out_ref):
    h = lax.dot_general(x_ref[...], wi_ref[...], (((1,), (1,)), ((), ())),
                        preferred_element_type=jnp.float32)
    h = jnp.maximum(h + bi_ref[...], 0.0)
    out_ref[...] = lax.dot_general(w8_ref[...], h, (((1,), (1,)), ((), ())),
                                   preferred_element_type=jnp.float32)


def _project(x, W_in, b_in, W8):
    return pl.pallas_call(
        _proj_body,
        out_shape=jax.ShapeDtypeStruct((8, N), jnp.float32),
    )(x, W_in, b_in.reshape(1, HIDDEN), W8)


_mesh = plsc.VectorSubcoreMesh(core_axis_name="c", subcore_axis_name="s")


def _loop(n, body):
    lax.fori_loop(0, n, lambda i, c: (body(i), 0)[1], 0)


_SC_OUT = (jax.ShapeDtypeStruct((NP,), jnp.float32),
           jax.ShapeDtypeStruct((NP,), jnp.float32))
_SC_SCRATCH = [
    pltpu.VMEM((NP,), jnp.float32),
    pltpu.VMEM((NP,), jnp.float32),
    pltpu.VMEM((NP,), jnp.float32),
    pltpu.VMEM((NP,), jnp.float32),
    pltpu.VMEM((EPT,), jnp.int32),
    pltpu.VMEM((EPT,), jnp.int32),
    pltpu.VMEM((SL,), jnp.float32),
    pltpu.VMEM((SL,), jnp.float32),
    pltpu.VMEM((SL,), jnp.float32),
    pltpu.VMEM((SL,), jnp.float32),
    pltpu.VMEM((SL,), jnp.float32),
    pltpu.VMEM(((K + 1) * L,), jnp.float32),
    pltpu.VMEM_SHARED((16, NP), jnp.float32),
    pltpu.VMEM_SHARED((16, NP), jnp.float32),
    pltpu.VMEM_SHARED((NP,), jnp.float32),
    pltpu.VMEM_SHARED((NP,), jnp.float32),
]


def _sc_body(z0_hbm, z1_hbm, row_hbm, col_hbm, w_hbm,
             out0_hbm, out1_hbm,
             pvf0, pvf1, acc0, acc1, rowb, colb,
             dbuf, obuf0, obuf1, tbuf, rbuf, wb,
             sacc0, sacc1, sv0, sv1):
    cid = lax.axis_index("c")
    t = lax.axis_index("s")
    base = t * SL

    zeros = jnp.zeros((L,), jnp.float32)
    ones = jnp.ones((L,), jnp.float32)
    NVP = NP // L
    SV = SL // L

    @pl.when(cid == 0)
    def _work():
        pltpu.sync_copy(w_hbm, wb)
        pltpu.sync_copy(row_hbm.at[pl.ds(t * EPT, EPT)], rowb)
        pltpu.sync_copy(col_hbm.at[pl.ds(t * EPT, EPT)], colb)
        pltpu.sync_copy(z0_hbm, pvf0.at[pl.ds(0, N)])
        pltpu.sync_copy(z1_hbm, pvf1.at[pl.ds(0, N)])

        def zpad(i):
            s = pl.ds(N + i * L, L)
            pvf0[s] = zeros
            pvf1[s] = zeros
        _loop((NP - N) // L, zpad)

        def slice_sum(dst, slots):
            def zz(i):
                for u in range(4):
                    dst[pl.ds((i * 4 + u) * L, L)] = zeros
            _loop(SV // 4, zz)
            for j in range(16):
                pltpu.sync_copy(slots.at[j].at[pl.ds(base, SL)], tbuf)

                def addv(i):
                    for u in range(4):
                        s = pl.ds((i * 4 + u) * L, L)
                        dst[s] = dst[s] + tbuf[s]
                _loop(SV // 4, addv)

        def zdeg(i):
            for u in range(4):
                acc0[pl.ds((i * 4 + u) * L, L)] = zeros
        _loop(NVP // 4, zdeg)

        def deg_vec(j):
            for u in range(4):
                plsc.addupdate_scatter(
                    acc0, [rowb[pl.ds((j * 4 + u) * L, L)]], ones)
        _loop(EPT // L // 4, deg_vec)
        pltpu.sync_copy(acc0, sacc0.at[t])
        plsc.subcore_barrier()
        slice_sum(rbuf, sacc0)

        def newton(i):
            s = pl.ds(i * L, L)
            d = rbuf[s] + 1.0
            y = plsc.bitcast(jnp.int32(0x5F3759DF)
                             - (plsc.bitcast(d, jnp.int32) >> 1), jnp.float32)
            y = y * (1.5 - 0.5 * d * y * y)
            y = y * (1.5 - 0.5 * d * y * y)
            y = y * (1.5 - 0.5 * d * y * y)
            dbuf[s] = y
        _loop(SV, newton)

        w0 = wb[pl.ds(0, L)]

        def init_slice(i):
            s = pl.ds(i * L, L)
            sg = pl.ds(base + i * L, L)
            z0v = pvf0[sg]
            z1v = pvf1[sg]
            obuf0[s] = w0 * z0v
            obuf1[s] = w0 * z1v
            tbuf[s] = z0v * dbuf[s]
            rbuf[s] = z1v * dbuf[s]
        _loop(SV, init_slice)
        pltpu.sync_copy(tbuf, sv0.at[pl.ds(base, SL)])
        pltpu.sync_copy(rbuf, sv1.at[pl.ds(base, SL)])
        plsc.subcore_barrier()
        pltpu.sync_copy(sv0, pvf0)
        pltpu.sync_copy(sv1, pvf1)

        def round_body(k):
            wk = wb[pl.ds(k * L, L)]

            def zacc(i):
                for u in range(4):
                    s = pl.ds((i * 4 + u) * L, L)
                    acc0[s] = zeros
                    acc1[s] = zeros
            _loop(NVP // 4, zacc)

            def edge_vec(j):
                for u in range(4):
                    s = pl.ds((j * 4 + u) * L, L)
                    rows = rowb[s]
                    cols = colb[s]
                    m0 = plsc.load_gather(pvf0, [cols])
                    plsc.addupdate_scatter(acc0, [rows], m0)
                    m1 = plsc.load_gather(pvf1, [cols])
                    plsc.addupdate_scatter(acc1, [rows], m1)
            _loop(EPT // L // 4, edge_vec)

            pltpu.sync_copy(acc0, sacc0.at[t])
            pltpu.sync_copy(acc1, sacc1.at[t])
            plsc.subcore_barrier()

            slice_sum(rbuf, sacc0)

            def upd0(i):
                s = pl.ds(i * L, L)
                sg = pl.ds(base + i * L, L)
                p = dbuf[s] * (rbuf[s] + pvf0[sg])
                obuf0[s] = obuf0[s] + wk * p
                tbuf[s] = p * dbuf[s]
            _loop(SV, upd0)
            pltpu.sync_copy(tbuf, sv0.at[pl.ds(base, SL)])

            slice_sum(rbuf, sacc1)

            def upd1(i):
                s = pl.ds(i * L, L)
                sg = pl.ds(base + i * L, L)
                p = dbuf[s] * (rbuf[s] + pvf1[sg])
                obuf1[s] = obuf1[s] + wk * p
                tbuf[s] = p * dbuf[s]
            _loop(SV, upd1)
            pltpu.sync_copy(tbuf, sv1.at[pl.ds(base, SL)])

            plsc.subcore_barrier()
            pltpu.sync_copy(sv0, pvf0)
            pltpu.sync_copy(sv1, pvf1)

        lax.fori_loop(1, K + 1, lambda k, c: (round_body(k), 0)[1], 0)

        pltpu.sync_copy(obuf0, out0_hbm.at[pl.ds(base, SL)])
        pltpu.sync_copy(obuf1, out1_hbm.at[pl.ds(base, SL)])


_sc_prop = pl.kernel(
    _sc_body,
    out_type=_SC_OUT,
    mesh=_mesh,
    compiler_params=pltpu.CompilerParams(needs_layout_passes=False),
    scratch_types=_SC_SCRATCH,
)


def kernel(x, edge_index, W_in, b_in, W_out, b_out, gpr_weights):
    W8 = jnp.zeros((8, HIDDEN), jnp.float32).at[:2].set(W_out)
    z8 = _project(x, W_in, b_in, W8)
    wspl = jnp.repeat(jax.nn.softmax(gpr_weights), L)
    o0, o1 = _sc_prop(z8[0], z8[1], edge_index[0], edge_index[1], wspl)
    return jnp.stack([o0[:N], o1[:N]], axis=1) + b_out


# speedup vs baseline: 1.0886x; 1.0886x over previous
import functools

import jax
import jax.numpy as jnp
from jax import lax
from jax.experimental import pallas as pl
from jax.experimental.pallas import tpu as pltpu
from jax.experimental.pallas import tpu_sc as plsc

N = 10000
E = 320000
D_IN = 128
HIDDEN = 128
K = 10
L = 16
NP = 10240
SL = NP // 16
EPT = E // 16


def _proj_body(x_ref, wi_ref, bi_ref, w8_ref, out_ref):
    h = lax.dot_general(x_ref[...], wi_ref[...], (((1,), (1,)), ((), ())),
                        preferred_element_type=jnp.float32)
    h = jnp.maximum(h + bi_ref[...], 0.0)
    out_ref[...] = lax.dot_general(w8_ref[...], h, (((1,), (1,)), ((), ())),
                                   preferred_element_type=jnp.float32)


def _project(x, W_in, b_in, W8):
    return pl.pallas_call(
        _proj_body,
        out_shape=jax.ShapeDtypeStruct((8, N), jnp.float32),
    )(x, W_in, b_in.reshape(1, HIDDEN), W8)


_mesh = plsc.VectorSubcoreMesh(core_axis_name="c", subcore_axis_name="s")


def _loop(n, body):
    lax.fori_loop(0, n, lambda i, c: (body(i), 0)[1], 0)


_SC_OUT = (jax.ShapeDtypeStruct((NP,), jnp.float32),
           jax.ShapeDtypeStruct((NP,), jnp.float32))
_SC_SCRATCH = [
    pltpu.VMEM((NP,), jnp.float32),
    pltpu.VMEM((NP,), jnp.float32),
    pltpu.VMEM((NP,), jnp.float32),
    pltpu.VMEM((NP,), jnp.float32),
    pltpu.VMEM((EPT,), jnp.int32),
    pltpu.VMEM((EPT,), jnp.int32),
    pltpu.VMEM((SL,), jnp.float32),
    pltpu.VMEM((SL,), jnp.float32),
    pltpu.VMEM((SL,), jnp.float32),
    pltpu.VMEM((SL,), jnp.float32),
    pltpu.VMEM((SL,), jnp.float32),
    pltpu.VMEM((SL,), jnp.float32),
    pltpu.VMEM(((K + 1) * L,), jnp.float32),
    pltpu.SemaphoreType.DMA,
    pltpu.SemaphoreType.DMA,
    pltpu.VMEM_SHARED((16, NP), jnp.float32),
    pltpu.VMEM_SHARED((16, NP), jnp.float32),
    pltpu.VMEM_SHARED((NP,), jnp.float32),
    pltpu.VMEM_SHARED((NP,), jnp.float32),
]


def _sc_body(z0_hbm, z1_hbm, row_hbm, col_hbm, w_hbm,
             out0_hbm, out1_hbm,
             pvf0, pvf1, acc0, acc1, rowb, colb,
             dbuf, obuf0, obuf1, tbuf, tbuf2, rbuf, wb, semA, semB,
             sacc0, sacc1, sv0, sv1):
    cid = lax.axis_index("c")
    t = lax.axis_index("s")
    base = t * SL

    zeros = jnp.zeros((L,), jnp.float32)
    ones = jnp.ones((L,), jnp.float32)
    NVP = NP // L
    SV = SL // L

    @pl.when(cid == 0)
    def _work():
        pltpu.sync_copy(w_hbm, wb)
        pltpu.sync_copy(row_hbm.at[pl.ds(t * EPT, EPT)], rowb)
        pltpu.sync_copy(col_hbm.at[pl.ds(t * EPT, EPT)], colb)
        pltpu.sync_copy(z0_hbm, pvf0.at[pl.ds(0, N)])
        pltpu.sync_copy(z1_hbm, pvf1.at[pl.ds(0, N)])

        def zpad(i):
            s = pl.ds(N + i * L, L)
            pvf0[s] = zeros
            pvf1[s] = zeros
        _loop((NP - N) // L, zpad)

        def slice_sum(dst, slots):
            def zz(i):
                for u in range(4):
                    dst[pl.ds((i * 4 + u) * L, L)] = zeros
            _loop(SV // 4, zz)
            bufs = (tbuf, tbuf2)
            sems = (semA, semB)
            h = pltpu.async_copy(slots.at[0].at[pl.ds(base, SL)], bufs[0],
                                 sems[0])
            for j in range(16):
                if j + 1 < 16:
                    hn = pltpu.async_copy(
                        slots.at[j + 1].at[pl.ds(base, SL)],
                        bufs[(j + 1) % 2], sems[(j + 1) % 2])
                h.wait()
                src = bufs[j % 2]

                def addv(i):
                    for u in range(4):
                        s = pl.ds((i * 4 + u) * L, L)
                        dst[s] = dst[s] + src[s]
                _loop(SV // 4, addv)
                if j + 1 < 16:
                    h = hn

        def zdeg(i):
            for u in range(4):
                acc0[pl.ds((i * 4 + u) * L, L)] = zeros
        _loop(NVP // 4, zdeg)

        def deg_vec(j):
            for u in range(4):
                plsc.addupdate_scatter(
                    acc0, [rowb[pl.ds((j * 4 + u) * L, L)]], ones)
        _loop(EPT // L // 4, deg_vec)
        pltpu.sync_copy(acc0, sacc0.at[t])
        plsc.subcore_barrier()
        slice_sum(rbuf, sacc0)

        def newton(i):
            s = pl.ds(i * L, L)
            d = rbuf[s] + 1.0
            y = plsc.bitcast(jnp.int32(0x5F3759DF)
                             - (plsc.bitcast(d, jnp.int32) >> 1), jnp.float32)
            y = y * (1.5 - 0.5 * d * y * y)
            y = y * (1.5 - 0.5 * d * y * y)
            y = y * (1.5 - 0.5 * d * y * y)
            dbuf[s] = y
        _loop(SV, newton)

        w0 = wb[pl.ds(0, L)]

        def init_slice(i):
            s = pl.ds(i * L, L)
            sg = pl.ds(base + i * L, L)
            z0v = pvf0[sg]
            z1v = pvf1[sg]
            obuf0[s] = w0 * z0v
            obuf1[s] = w0 * z1v
            tbuf[s] = z0v * dbuf[s]
            rbuf[s] = z1v * dbuf[s]
        _loop(SV, init_slice)
        pltpu.sync_copy(tbuf, sv0.at[pl.ds(base, SL)])
        pltpu.sync_copy(rbuf, sv1.at[pl.ds(base, SL)])
        plsc.subcore_barrier()
        pltpu.sync_copy(sv0, pvf0)
        pltpu.sync_copy(sv1, pvf1)

        def round_body(k):
            wk = wb[pl.ds(k * L, L)]

            def zacc(i):
                for u in range(4):
                    s = pl.ds((i * 4 + u) * L, L)
                    acc0[s] = zeros
                    acc1[s] = zeros
            _loop(NVP // 4, zacc)

            def edge_vec(j):
                for u in range(2):
                    s = pl.ds((j * 2 + u) * L, L)
                    rows = rowb[s]
                    cols = colb[s]
                    m0 = plsc.load_gather(pvf0, [cols])
                    plsc.addupdate_scatter(acc0, [rows], m0)
                    m1 = plsc.load_gather(pvf1, [cols])
                    plsc.addupdate_scatter(acc1, [rows], m1)
            _loop(EPT // L // 2, edge_vec)

            h0 = pltpu.async_copy(acc0, sacc0.at[t], semA)
            h1 = pltpu.async_copy(acc1, sacc1.at[t], semB)
            h0.wait()
            h1.wait()
            plsc.subcore_barrier()

            slice_sum(rbuf, sacc0)

            def upd0(i):
                s = pl.ds(i * L, L)
                sg = pl.ds(base + i * L, L)
                p = dbuf[s] * (rbuf[s] + pvf0[sg])
                obuf0[s] = obuf0[s] + wk * p
                tbuf[s] = p * dbuf[s]
            _loop(SV, upd0)
            pltpu.sync_copy(tbuf, sv0.at[pl.ds(base, SL)])

            slice_sum(rbuf, sacc1)

            def upd1(i):
                s = pl.ds(i * L, L)
                sg = pl.ds(base + i * L, L)
                p = dbuf[s] * (rbuf[s] + pvf1[sg])
                obuf1[s] = obuf1[s] + wk * p
                tbuf[s] = p * dbuf[s]
            _loop(SV, upd1)
            pltpu.sync_copy(tbuf, sv1.at[pl.ds(base, SL)])

            plsc.subcore_barrier()
            p0 = pltpu.async_copy(sv0, pvf0, semA)
            p1 = pltpu.async_copy(sv1, pvf1, semB)
            p0.wait()
            p1.wait()

        lax.fori_loop(1, K + 1, lambda k, c: (round_body(k), 0)[1], 0)

        pltpu.sync_copy(obuf0, out0_hbm.at[pl.ds(base, SL)])
        pltpu.sync_copy(obuf1, out1_hbm.at[pl.ds(base, SL)])


_sc_prop = pl.kernel(
    _sc_body,
    out_type=_SC_OUT,
    mesh=_mesh,
    compiler_params=pltpu.CompilerParams(needs_layout_passes=False),
    scratch_types=_SC_SCRATCH,
)


def kernel(x, edge_index, W_in, b_in, W_out, b_out, gpr_weights):
    W8 = jnp.zeros((8, HIDDEN), jnp.float32).at[:2].set(W_out)
    z8 = _project(x, W_in, b_in, W8)
    wspl = jnp.repeat(jax.nn.softmax(gpr_weights), L)
    o0, o1 = _sc_prop(z8[0], z8[1], edge_index[0], edge_index[1], wspl)
    return jnp.stack([o0[:N], o1[:N]], axis=1) + b_out
